# Initial kernel scaffold; baseline (speedup 1.0000x reference)
import jax
import jax.numpy as jnp
from jax.experimental import pallas as pl


def kernel(x_nchw, weight_oihw, bias, gamma, beta):
    raise NotImplementedError("write your pallas kernel here")


# speedup vs baseline: 2.9983x; 2.9983x over previous
import functools

import jax
import jax.numpy as jnp
from jax.experimental import pallas as pl
from jax.experimental.pallas import tpu as pltpu


def _tap_cols(x_f32, H, W, KH, KW):
    HW = H * W
    C_in = x_f32.shape[0]
    xb = x_f32.astype(jnp.bfloat16)
    hw = jax.lax.broadcasted_iota(jnp.int32, (1, HW), 1)
    wcol = jax.lax.rem(hw, W)
    hrow = jax.lax.div(hw, W)
    cH, cW = (KH - 1) // 2, (KW - 1) // 2

    groups = []
    for kh in range(KH):
        dh = kh - cH
        taps = []
        for kw in range(KW):
            dw = kw - cW
            s = dh * W + dw
            t = pltpu.roll(xb, (-s) % HW, axis=1) if s != 0 else xb
            if dh != 0 or dw != 0:
                m = jnp.full((1, HW), True)
                if dh != 0:
                    m = m & (hrow + dh >= 0) & (hrow + dh < H)
                if dw != 0:
                    m = m & (wcol + dw >= 0) & (wcol + dw < W)
                t = jnp.where(m, t, jnp.bfloat16(0))
            taps.append(t)
        groups.append(jnp.concatenate(taps, axis=0))
    return groups


def _conv_stats_kernel(x_ref, w0_ref, w1_ref, w2_ref, stats_ref, *,
                       H, W, KH, KW):
    groups = _tap_cols(x_ref[0], H, W, KH, KW)
    wrefs = (w0_ref, w1_ref, w2_ref)
    acc = jnp.dot(wrefs[0][...], groups[0],
                  preferred_element_type=jnp.float32)
    for k in range(1, KH):
        acc = acc + jnp.dot(wrefs[k][...], groups[k],
                            preferred_element_type=jnp.float32)
    s = jnp.sum(acc, axis=1)[None, :]
    ss = jnp.sum(acc * acc, axis=1)[None, :]
    stats_ref[0] = jnp.concatenate([s, ss], axis=0)


def _bn_apply_kernel(x_ref, w0_ref, w1_ref, w2_ref, o_ref, *,
                     H, W, KH, KW):
    groups = _tap_cols(x_ref[0], H, W, KH, KW)
    HW = H * W
    groups[KH - 1] = jnp.concatenate(
        [groups[KH - 1], jnp.ones((1, HW), jnp.bfloat16)], axis=0)
    wrefs = (w0_ref, w1_ref, w2_ref)
    acc = jnp.dot(wrefs[0][...], groups[0],
                  preferred_element_type=jnp.float32)
    for k in range(1, KH):
        acc = acc + jnp.dot(wrefs[k][...], groups[k],
                            preferred_element_type=jnp.float32)
    o_ref[0] = jnp.maximum(acc, 0.0)


def kernel(x_nchw, weight_oihw, bias, gamma, beta):
    del bias
    N, C_in, H, W = x_nchw.shape
    C_out, _, KH, KW = weight_oihw.shape
    HW = H * W
    eps = 1e-5

    xf = x_nchw.reshape(N, C_in, HW)
    w2 = jnp.transpose(weight_oihw, (0, 2, 3, 1)).reshape(C_out, KH * KW * C_in)
    gk = KW * C_in
    wg = [w2[:, k * gk:(k + 1) * gk].astype(jnp.bfloat16) for k in range(KH)]

    cp = pltpu.CompilerParams(dimension_semantics=("parallel",),
                              vmem_limit_bytes=64 * 1024 * 1024)

    stats = pl.pallas_call(
        functools.partial(_conv_stats_kernel, H=H, W=W, KH=KH, KW=KW),
        out_shape=jax.ShapeDtypeStruct((N, 2, C_out), jnp.float32),
        grid=(N,),
        in_specs=[
            pl.BlockSpec((1, C_in, HW), lambda n: (n, 0, 0)),
            pl.BlockSpec((C_out, gk), lambda n: (0, 0)),
            pl.BlockSpec((C_out, gk), lambda n: (0, 0)),
            pl.BlockSpec((C_out, gk), lambda n: (0, 0)),
        ],
        out_specs=pl.BlockSpec((1, 2, C_out), lambda n: (n, 0, 0)),
        compiler_params=cp,
    )(xf, wg[0], wg[1], wg[2])

    count = jnp.float32(N * HW)
    mean = jnp.sum(stats[:, 0, :], axis=0) / count
    var = jnp.maximum(jnp.sum(stats[:, 1, :], axis=0) / count - mean * mean,
                      0.0)
    inv_std = jax.lax.rsqrt(var + eps)
    g32 = gamma.astype(jnp.float32)
    scale = g32 * inv_std
    shift = beta.astype(jnp.float32) - mean * scale

    w2s = w2 * scale[:, None]
    wgs = [w2s[:, k * gk:(k + 1) * gk] for k in range(KH)]
    wgs[KH - 1] = jnp.concatenate([wgs[KH - 1], shift[:, None]], axis=1)
    wgs = [w.astype(jnp.bfloat16) for w in wgs]

    y = pl.pallas_call(
        functools.partial(_bn_apply_kernel, H=H, W=W, KH=KH, KW=KW),
        out_shape=jax.ShapeDtypeStruct((N, C_out, HW), jnp.float32),
        grid=(N,),
        in_specs=[
            pl.BlockSpec((1, C_in, HW), lambda n: (n, 0, 0)),
            pl.BlockSpec((C_out, gk), lambda n: (0, 0)),
            pl.BlockSpec((C_out, gk), lambda n: (0, 0)),
            pl.BlockSpec((C_out, gk + 1), lambda n: (0, 0)),
        ],
        out_specs=pl.BlockSpec((1, C_out, HW), lambda n: (n, 0, 0)),
        compiler_params=cp,
    )(xf, wgs[0], wgs[1], wgs[2])

    return y.reshape(N, C_out, H, W)
